# baseline (speedup 1.0000x reference)
import jax
import jax.numpy as jnp
from jax import lax
from jax.experimental import pallas as pl
from jax.experimental.pallas import tpu as pltpu
from jax.experimental.pallas import tpu_sc as plsc

N = 320000
D = 128
S = 10000

NROW = N // 128
NROWP = 2560

TC_T = 40960
TC_GRID = (N + TC_T - 1) // TC_T

L = 16
NSUB = 16
ROWS = NROWP // NSUB
SEG = 1024
SEGR = SEG // 128
SPAD = NSUB * SEG
TRASH = SPAD


def _tc_body(x_ref, w1_ref, b1_ref, w2_ref, o_ref):
    h = jnp.maximum(
        jnp.dot(x_ref[...], w1_ref[...], preferred_element_type=jnp.float32)
        + b1_ref[...], 0.0)
    v = jnp.sum(h * w2_ref[...], axis=1)
    o_ref[...] = v.reshape(o_ref.shape)


def _tc_v(x, W1, b1, W2):
    return pl.pallas_call(
        _tc_body,
        grid=(TC_GRID,),
        in_specs=[
            pl.BlockSpec((TC_T, D), lambda i: (i, 0)),
            pl.BlockSpec((D, D), lambda i: (0, 0)),
            pl.BlockSpec((1, D), lambda i: (0, 0)),
            pl.BlockSpec((1, D), lambda i: (0, 0)),
        ],
        out_specs=pl.BlockSpec((TC_T // 128, 128), lambda i: (i, 0)),
        out_shape=jax.ShapeDtypeStruct((NROWP, 128), jnp.float32),
    )(x, W1, b1.reshape(1, D), W2.reshape(1, D))


def _sca_body(idp, cnts_out, ranks_out, k_out,
              cnts_sh, idv, onesr, zbuf, call_, cbuf, rbuf, k16, sem):
    wid = lax.axis_index("s")
    sbase = wid * SEG
    rbase = wid * ROWS

    for i in range(SEG // L):
        zbuf[pl.ds(i * L, L)] = jnp.zeros((L,), jnp.float32)
    for i in range(128 // L):
        onesr[pl.ds(i * L, L)] = jnp.ones((L,), jnp.float32)
    pltpu.sync_copy(zbuf, cnts_sh.at[pl.ds(sbase, SEG)])
    pltpu.sync_copy(idp.at[pl.ds(rbase, ROWS)], idv)
    plsc.subcore_barrier()

    def sa_body(r, carry):
        pltpu.async_copy(onesr, cnts_sh.at[idv.at[r]], sem, add=True)
        return carry

    lax.fori_loop(0, ROWS, sa_body, 0)
    pltpu.make_async_copy(idp.at[pl.ds(rbase, ROWS)], idv, sem).wait()
    plsc.subcore_barrier()

    pltpu.sync_copy(cnts_sh.at[pl.ds(0, SPAD)], call_)
    iota = lax.iota(jnp.int32, L)

    def scan_body(i, carry):
        off_acc, k_acc = carry
        c = call_[pl.ds(i * L, L)]
        presf = jnp.where(c > 0.0, 1.0, 0.0)
        pos = i * L + iota
        off_acc = off_acc + jnp.where(pos < sbase, presf, 0.0)
        return off_acc, k_acc + presf

    zv = jnp.zeros((L,), jnp.float32)
    off_acc, k_acc = lax.fori_loop(0, SPAD // L, scan_body, (zv, zv))
    offset = jnp.sum(off_acc)
    K = jnp.sum(k_acc)

    pltpu.sync_copy(cnts_sh.at[pl.ds(sbase, SEG)], cbuf)
    off = offset
    for i in range(SEG // L):
        cv = cbuf[pl.ds(i * L, L)]
        pres = cv > 0.0
        presf = jnp.where(pres, 1.0, 0.0)
        incl = plsc.cumsum(presf)
        rank = jnp.where(pres, (off + incl - presf).astype(jnp.int32), TRASH)
        row, col = (i * L) // 128, (i * L) % 128
        rbuf[row, pl.ds(col, L)] = rank
        off = off + jnp.sum(presf)

    pltpu.sync_copy(cbuf, cnts_out.at[pl.ds(sbase, SEG)])
    pltpu.sync_copy(rbuf, ranks_out.at[pl.ds(wid * SEGR, SEGR)])

    @pl.when(wid == 0)
    def _():
        for i in range(128 // L):
            k16[pl.ds(i * L, L)] = jnp.full((L,), K, jnp.float32)
        pltpu.sync_copy(k16, k_out)


def _sc_counts(idp):
    mesh = plsc.VectorSubcoreMesh(
        core_axis_name="c", subcore_axis_name="s", num_cores=1)
    scratch = [
        pltpu.VMEM_SHARED((SPAD + L,), jnp.float32),
        pltpu.VMEM((ROWS, 128), jnp.int32),
        pltpu.VMEM((128,), jnp.float32),
        pltpu.VMEM((SEG,), jnp.float32),
        pltpu.VMEM((SPAD,), jnp.float32),
        pltpu.VMEM((SEG,), jnp.float32),
        pltpu.VMEM((SEGR, 128), jnp.int32),
        pltpu.VMEM((128,), jnp.float32),
        pltpu.SemaphoreType.DMA,
    ]
    f = pl.kernel(
        _sca_body,
        out_type=(
            jax.ShapeDtypeStruct((SPAD,), jnp.float32),
            jax.ShapeDtypeStruct((SPAD // 128, 128), jnp.int32),
            jax.ShapeDtypeStruct((128,), jnp.float32),
        ),
        mesh=mesh,
        scratch_types=scratch,
        compiler_params=pltpu.CompilerParams(needs_layout_passes=False),
    )
    return f(idp)


def _scb_body(vp, idp, cnts, ranks, kk, b2p, out_hbm,
              sums_sh, out_sh,
              idv, vv, zbuf, sbuf, cbuf, mbuf, rbuf, obuf, fl16, b2v, kv,
              sem):
    wid = lax.axis_index("s")
    sbase = wid * SEG
    rbase = wid * ROWS

    for i in range(SEG // L):
        zbuf[pl.ds(i * L, L)] = jnp.zeros((L,), jnp.float32)
    pltpu.sync_copy(zbuf, sums_sh.at[pl.ds(sbase, SEG)])
    pltpu.sync_copy(idp.at[pl.ds(rbase, ROWS)], idv)
    pltpu.sync_copy(vp.at[pl.ds(rbase, ROWS)], vv)
    pltpu.sync_copy(b2p, b2v)
    pltpu.sync_copy(kk.at[pl.ds(0, L)], kv)
    pltpu.sync_copy(cnts.at[pl.ds(sbase, SEG)], cbuf)
    pltpu.sync_copy(ranks.at[pl.ds(wid * SEGR, SEGR)], rbuf)
    plsc.subcore_barrier()

    def sa_body(r, carry):
        pltpu.async_copy(vv.at[r], sums_sh.at[idv.at[r]], sem, add=True)
        return carry

    lax.fori_loop(0, ROWS, sa_body, 0)
    pltpu.make_async_copy(vp.at[pl.ds(rbase, ROWS)], vv, sem).wait()
    plsc.subcore_barrier()

    pltpu.sync_copy(sums_sh.at[pl.ds(sbase, SEG)], sbuf)
    b2vec = b2v[...]
    for i in range(SEG // L):
        sv = sbuf[pl.ds(i * L, L)]
        cv = cbuf[pl.ds(i * L, L)]
        mean = sv / jnp.maximum(cv, 1.0) + b2vec
        row, col = (i * L) // 128, (i * L) % 128
        mbuf[row, pl.ds(col, L)] = mean
    for r in range(SEGR):
        pltpu.sync_copy(mbuf.at[r], out_sh.at[rbuf.at[r]])
    plsc.subcore_barrier()

    pltpu.sync_copy(out_sh.at[pl.ds(0, L)], fl16)
    fill = jnp.full((L,), fl16[...][0], jnp.float32)
    K = kv[...][0]
    iota = lax.iota(jnp.int32, L)
    SFLOOR = (S // SEG) * SEG
    STAIL = S - SFLOOR

    @pl.when(sbase < S)
    def _():
        pltpu.sync_copy(out_sh.at[pl.ds(sbase, SEG)], obuf)
        for i in range(SEG // L):
            cur = obuf[pl.ds(i * L, L)]
            posf = (sbase + i * L + iota).astype(jnp.float32)
            obuf[pl.ds(i * L, L)] = jnp.where(posf >= K, fill, cur)

    @pl.when(sbase + SEG <= S)
    def _():
        pltpu.sync_copy(obuf, out_hbm.at[pl.ds(sbase, SEG)])

    @pl.when(jnp.logical_and(sbase < S, sbase + SEG > S))
    def _():
        pltpu.sync_copy(obuf.at[pl.ds(0, STAIL)],
                        out_hbm.at[pl.ds(sbase, STAIL)])


def _sc_means(vp, idp, cnts, ranks, kk, b2p):
    mesh = plsc.VectorSubcoreMesh(
        core_axis_name="c", subcore_axis_name="s", num_cores=1)
    scratch = [
        pltpu.VMEM_SHARED((SPAD + L,), jnp.float32),
        pltpu.VMEM_SHARED((SPAD + L,), jnp.float32),
        pltpu.VMEM((ROWS, 128), jnp.int32),
        pltpu.VMEM((ROWS, 128), jnp.float32),
        pltpu.VMEM((SEG,), jnp.float32),
        pltpu.VMEM((SEG,), jnp.float32),
        pltpu.VMEM((SEG,), jnp.float32),
        pltpu.VMEM((SEGR, 128), jnp.float32),
        pltpu.VMEM((SEGR, 128), jnp.int32),
        pltpu.VMEM((SEG,), jnp.float32),
        pltpu.VMEM((L,), jnp.float32),
        pltpu.VMEM((L,), jnp.float32),
        pltpu.VMEM((L,), jnp.float32),
        pltpu.SemaphoreType.DMA,
    ]
    f = pl.kernel(
        _scb_body,
        out_type=jax.ShapeDtypeStruct((S,), jnp.float32),
        mesh=mesh,
        scratch_types=scratch,
        compiler_params=pltpu.CompilerParams(needs_layout_passes=False),
    )
    return f(vp, idp, cnts, ranks, kk, b2p)


def kernel(x, ids, W1, b1, W2, b2):
    vp = _tc_v(x, W1, b1, W2)
    idp = jnp.pad(ids.astype(jnp.int32), (0, NROWP * 128 - N),
                  constant_values=TRASH).reshape(NROWP, 128)
    cnts, ranks, kk = _sc_counts(idp)
    b2p = jnp.broadcast_to(b2, (L,))
    o = _sc_means(vp, idp, cnts, ranks, kk, b2p)
    return o.reshape(S, 1)


# speedup vs baseline: 1.0041x; 1.0041x over previous
import jax
import jax.numpy as jnp
from jax import lax
from jax.experimental import pallas as pl
from jax.experimental.pallas import tpu as pltpu
from jax.experimental.pallas import tpu_sc as plsc

N = 320000
D = 128
S = 10000

NROW = N // 128
NROWP = 2560

TC_T = 32768
TC_GRID = (N + TC_T - 1) // TC_T

L = 16
NSUB = 16
ROWS = NROWP // NSUB
SEG = 1024
SEGR = SEG // 128
SPAD = NSUB * SEG
TRASH = SPAD


def _tc_body(x_ref, w1_ref, b1_ref, w2_ref, o_ref):
    h = jnp.maximum(
        jnp.dot(x_ref[...], w1_ref[...], preferred_element_type=jnp.float32)
        + b1_ref[...], 0.0)
    v = jnp.sum(h * w2_ref[...], axis=1)
    o_ref[...] = v.reshape(o_ref.shape)


def _tc_v(x, W1, b1, W2):
    return pl.pallas_call(
        _tc_body,
        grid=(TC_GRID,),
        in_specs=[
            pl.BlockSpec((TC_T, D), lambda i: (i, 0)),
            pl.BlockSpec((D, D), lambda i: (0, 0)),
            pl.BlockSpec((1, D), lambda i: (0, 0)),
            pl.BlockSpec((1, D), lambda i: (0, 0)),
        ],
        out_specs=pl.BlockSpec((TC_T // 128, 128), lambda i: (i, 0)),
        out_shape=jax.ShapeDtypeStruct((NROWP, 128), jnp.float32),
    )(x, W1, b1.reshape(1, D), W2.reshape(1, D))


def _sca_body(idp, cnts_out, ranks_out, k_out,
              cnts_sh, idv, onesr, zbuf, call_, cbuf, rbuf, k16, sem):
    wid = lax.axis_index("s")
    sbase = wid * SEG
    rbase = wid * ROWS

    for i in range(SEG // L):
        zbuf[pl.ds(i * L, L)] = jnp.zeros((L,), jnp.float32)
    for i in range(128 // L):
        onesr[pl.ds(i * L, L)] = jnp.ones((L,), jnp.float32)
    pltpu.sync_copy(zbuf, cnts_sh.at[pl.ds(sbase, SEG)])
    pltpu.sync_copy(idp.at[pl.ds(rbase, ROWS)], idv)
    plsc.subcore_barrier()

    def sa_body(r, carry):
        pltpu.async_copy(onesr, cnts_sh.at[idv.at[r]], sem, add=True)
        return carry

    lax.fori_loop(0, ROWS, sa_body, 0)
    pltpu.make_async_copy(idp.at[pl.ds(rbase, ROWS)], idv, sem).wait()
    plsc.subcore_barrier()

    pltpu.sync_copy(cnts_sh.at[pl.ds(0, SPAD)], call_)
    iota = lax.iota(jnp.int32, L)

    def scan_body(i, carry):
        off_acc, k_acc = carry
        c = call_[pl.ds(i * L, L)]
        presf = jnp.where(c > 0.0, 1.0, 0.0)
        pos = i * L + iota
        off_acc = off_acc + jnp.where(pos < sbase, presf, 0.0)
        return off_acc, k_acc + presf

    zv = jnp.zeros((L,), jnp.float32)
    off_acc, k_acc = lax.fori_loop(0, SPAD // L, scan_body, (zv, zv))
    offset = jnp.sum(off_acc)
    K = jnp.sum(k_acc)

    pltpu.sync_copy(cnts_sh.at[pl.ds(sbase, SEG)], cbuf)
    off = offset
    for i in range(SEG // L):
        cv = cbuf[pl.ds(i * L, L)]
        pres = cv > 0.0
        presf = jnp.where(pres, 1.0, 0.0)
        incl = plsc.cumsum(presf)
        rank = jnp.where(pres, (off + incl - presf).astype(jnp.int32), TRASH)
        row, col = (i * L) // 128, (i * L) % 128
        rbuf[row, pl.ds(col, L)] = rank
        off = off + jnp.sum(presf)

    pltpu.sync_copy(cbuf, cnts_out.at[pl.ds(sbase, SEG)])
    pltpu.sync_copy(rbuf, ranks_out.at[pl.ds(wid * SEGR, SEGR)])

    @pl.when(wid == 0)
    def _():
        for i in range(128 // L):
            k16[pl.ds(i * L, L)] = jnp.full((L,), K, jnp.float32)
        pltpu.sync_copy(k16, k_out)


def _sc_counts(idp):
    mesh = plsc.VectorSubcoreMesh(
        core_axis_name="c", subcore_axis_name="s", num_cores=1)
    scratch = [
        pltpu.VMEM_SHARED((SPAD + L,), jnp.float32),
        pltpu.VMEM((ROWS, 128), jnp.int32),
        pltpu.VMEM((128,), jnp.float32),
        pltpu.VMEM((SEG,), jnp.float32),
        pltpu.VMEM((SPAD,), jnp.float32),
        pltpu.VMEM((SEG,), jnp.float32),
        pltpu.VMEM((SEGR, 128), jnp.int32),
        pltpu.VMEM((128,), jnp.float32),
        pltpu.SemaphoreType.DMA,
    ]
    f = pl.kernel(
        _sca_body,
        out_type=(
            jax.ShapeDtypeStruct((SPAD,), jnp.float32),
            jax.ShapeDtypeStruct((SPAD // 128, 128), jnp.int32),
            jax.ShapeDtypeStruct((128,), jnp.float32),
        ),
        mesh=mesh,
        scratch_types=scratch,
        compiler_params=pltpu.CompilerParams(needs_layout_passes=False),
    )
    return f(idp)


def _scb_body(vp, idp, cnts, ranks, kk, b2p, out_hbm,
              sums_sh, out_sh,
              idv, vv, zbuf, sbuf, cbuf, mbuf, rbuf, obuf, fl16, b2v, kv,
              sem):
    wid = lax.axis_index("s")
    sbase = wid * SEG
    rbase = wid * ROWS

    for i in range(SEG // L):
        zbuf[pl.ds(i * L, L)] = jnp.zeros((L,), jnp.float32)
    pltpu.sync_copy(zbuf, sums_sh.at[pl.ds(sbase, SEG)])
    pltpu.sync_copy(idp.at[pl.ds(rbase, ROWS)], idv)
    pltpu.sync_copy(vp.at[pl.ds(rbase, ROWS)], vv)
    pltpu.sync_copy(b2p, b2v)
    pltpu.sync_copy(kk.at[pl.ds(0, L)], kv)
    pltpu.sync_copy(cnts.at[pl.ds(sbase, SEG)], cbuf)
    pltpu.sync_copy(ranks.at[pl.ds(wid * SEGR, SEGR)], rbuf)
    plsc.subcore_barrier()

    def sa_body(r, carry):
        pltpu.async_copy(vv.at[r], sums_sh.at[idv.at[r]], sem, add=True)
        return carry

    lax.fori_loop(0, ROWS, sa_body, 0)
    pltpu.make_async_copy(vp.at[pl.ds(rbase, ROWS)], vv, sem).wait()
    plsc.subcore_barrier()

    pltpu.sync_copy(sums_sh.at[pl.ds(sbase, SEG)], sbuf)
    b2vec = b2v[...]
    for i in range(SEG // L):
        sv = sbuf[pl.ds(i * L, L)]
        cv = cbuf[pl.ds(i * L, L)]
        mean = sv / jnp.maximum(cv, 1.0) + b2vec
        row, col = (i * L) // 128, (i * L) % 128
        mbuf[row, pl.ds(col, L)] = mean
    for r in range(SEGR):
        pltpu.sync_copy(mbuf.at[r], out_sh.at[rbuf.at[r]])
    plsc.subcore_barrier()

    pltpu.sync_copy(out_sh.at[pl.ds(0, L)], fl16)
    fill = jnp.full((L,), fl16[...][0], jnp.float32)
    K = kv[...][0]
    iota = lax.iota(jnp.int32, L)
    SFLOOR = (S // SEG) * SEG
    STAIL = S - SFLOOR

    @pl.when(sbase < S)
    def _():
        pltpu.sync_copy(out_sh.at[pl.ds(sbase, SEG)], obuf)
        for i in range(SEG // L):
            cur = obuf[pl.ds(i * L, L)]
            posf = (sbase + i * L + iota).astype(jnp.float32)
            obuf[pl.ds(i * L, L)] = jnp.where(posf >= K, fill, cur)

    @pl.when(sbase + SEG <= S)
    def _():
        pltpu.sync_copy(obuf, out_hbm.at[pl.ds(sbase, SEG)])

    @pl.when(jnp.logical_and(sbase < S, sbase + SEG > S))
    def _():
        pltpu.sync_copy(obuf.at[pl.ds(0, STAIL)],
                        out_hbm.at[pl.ds(sbase, STAIL)])


def _sc_means(vp, idp, cnts, ranks, kk, b2p):
    mesh = plsc.VectorSubcoreMesh(
        core_axis_name="c", subcore_axis_name="s", num_cores=1)
    scratch = [
        pltpu.VMEM_SHARED((SPAD + L,), jnp.float32),
        pltpu.VMEM_SHARED((SPAD + L,), jnp.float32),
        pltpu.VMEM((ROWS, 128), jnp.int32),
        pltpu.VMEM((ROWS, 128), jnp.float32),
        pltpu.VMEM((SEG,), jnp.float32),
        pltpu.VMEM((SEG,), jnp.float32),
        pltpu.VMEM((SEG,), jnp.float32),
        pltpu.VMEM((SEGR, 128), jnp.float32),
        pltpu.VMEM((SEGR, 128), jnp.int32),
        pltpu.VMEM((SEG,), jnp.float32),
        pltpu.VMEM((L,), jnp.float32),
        pltpu.VMEM((L,), jnp.float32),
        pltpu.VMEM((L,), jnp.float32),
        pltpu.SemaphoreType.DMA,
    ]
    f = pl.kernel(
        _scb_body,
        out_type=jax.ShapeDtypeStruct((S,), jnp.float32),
        mesh=mesh,
        scratch_types=scratch,
        compiler_params=pltpu.CompilerParams(needs_layout_passes=False),
    )
    return f(vp, idp, cnts, ranks, kk, b2p)


def kernel(x, ids, W1, b1, W2, b2):
    vp = _tc_v(x, W1, b1, W2)
    idp = jnp.pad(ids.astype(jnp.int32), (0, NROWP * 128 - N),
                  constant_values=TRASH).reshape(NROWP, 128)
    cnts, ranks, kk = _sc_counts(idp)
    b2p = jnp.broadcast_to(b2, (L,))
    o = _sc_means(vp, idp, cnts, ranks, kk, b2p)
    return o.reshape(S, 1)
